# baseline (speedup 1.0000x reference)
import functools

import jax
import jax.numpy as jnp
from jax import lax
from jax.experimental import pallas as pl
from jax.experimental.pallas import tpu as pltpu
from jax.experimental.pallas import tpu_sc as plsc

_N = 10000
_E = 320000
_D = 128
_CP = 144
_NC = 2
_NS = 16
_NW = _NC * _NS
_EPW = _E // _NW
_K = 80
_NCHUNK = _EPW // _K
_NTRIPLE = (_NCHUNK - 5) // 3
_SCALE_UNROLL = 4
_NP = 10240
_RPT = _NP // _NS
_LANES = 16


def _proj_body(x_ref, w_ref, as_ref, ad_ref, hext_ref, av_ref, dv_ref):
    x = x_ref[...]
    h = lax.dot_general(x, w_ref[...], (((1,), (1,)), ((), ())),
                        preferred_element_type=jnp.float32)
    b = h.shape[0]
    tail = (lax.broadcasted_iota(jnp.int32, (b, _CP - _D), 1) == 0)
    hext_ref[...] = jnp.concatenate([h, tail.astype(jnp.float32)], axis=1)
    av_ref[...] = jnp.sum(h * as_ref[...], axis=1, keepdims=True)
    dv_ref[...] = jnp.sum(h * ad_ref[...], axis=1, keepdims=True)


def _edge_body(hext_hbm, asrc_hbm, adst_hbm, src_hbm, dst_hbm, zero_hbm,
               outp_hbm, src_a, dst_a, av_a, bv_a, rows_a, src_b, dst_b, av_b,
               bv_b, rows_b, src_c3, dst_c3, av_c3, bv_c3, rows_c3, wbuf,
               asrc_s, adst_s, acc, sem_ra, sem_va, sem_sa, sem_rb, sem_vb,
               sem_sb, sem_rc, sem_vc, sem_sc):
    cid = lax.axis_index("c")
    sid = lax.axis_index("s")
    wid = cid * _NS + sid
    pltpu.sync_copy(zero_hbm, acc.at[pl.ds(sid * _RPT, _RPT)])
    @pl.when(sid == 0)
    def _stage():
        pltpu.sync_copy(asrc_hbm, asrc_s)
        pltpu.sync_copy(adst_hbm, adst_s)
    plsc.subcore_barrier()

    base = wid * _EPW
    marker = (lax.iota(jnp.int32, 16) == 0).astype(jnp.float32)

    bufs = (
        (src_a, dst_a, av_a, bv_a, rows_a, sem_ra, sem_va, sem_sa),
        (src_b, dst_b, av_b, bv_b, rows_b, sem_rb, sem_vb, sem_sb),
        (src_c3, dst_c3, av_c3, bv_c3, rows_c3, sem_rc, sem_vc, sem_sc),
    )

    def fetch(j, buf, first=False):
        src_c, dst_c, av_c, bv_c, rows, sem_r, sem_v, sem_s = buf
        if not first:
            pltpu.make_async_copy(rows, acc.at[dst_c], sem_s).wait()
        off = base + j * _K
        pltpu.sync_copy(src_hbm.at[pl.ds(off, _K)], src_c)
        pltpu.sync_copy(dst_hbm.at[pl.ds(off, _K)], dst_c)
        pltpu.async_copy(hext_hbm.at[src_c], rows, sem_r)
        pltpu.async_copy(asrc_s.at[src_c], av_c, sem_v)
        pltpu.async_copy(adst_s.at[dst_c], bv_c, sem_v)

    def process(buf):
        src_c, dst_c, av_c, bv_c, rows, sem_r, sem_v, sem_s = buf
        pltpu.make_async_copy(asrc_s.at[src_c], av_c, sem_v).wait()
        pltpu.make_async_copy(adst_s.at[dst_c], bv_c, sem_v).wait()
        for g in range(_K // _LANES):
            sl = pl.ds(g * _LANES, _LANES)
            a = av_c[sl] + bv_c[sl]
            a = jnp.where(a >= 0, a, 0.2 * a)
            wbuf[sl] = jnp.exp(a)
        pltpu.make_async_copy(hext_hbm.at[src_c], rows, sem_r).wait()

        def scale(i, c2):
            for u in range(_SCALE_UNROLL):
                e = i * _SCALE_UNROLL + u
                w16 = plsc.load_gather(
                    wbuf, [jnp.full((_LANES,), e, jnp.int32)])
                for v in range(_D // _LANES):
                    sl = pl.ds(v * _LANES, _LANES)
                    rows[e, sl] = rows[e, sl] * w16
                rows[e, pl.ds(_D, _LANES)] = w16 * marker
            return c2

        lax.fori_loop(0, _K // _SCALE_UNROLL, scale, 0)
        pltpu.async_copy(rows, acc.at[dst_c], sem_s, add=True)

    fetch(0, bufs[0], first=True)
    fetch(1, bufs[1], first=True)
    fetch(2, bufs[2], first=True)
    process(bufs[0])
    fetch(3, bufs[0])
    process(bufs[1])
    fetch(4, bufs[1])
    process(bufs[2])

    def triple(t, carry):
        j0 = 3 * t
        fetch(j0 + 2, bufs[2])
        process(bufs[0])
        fetch(j0 + 3, bufs[0])
        process(bufs[1])
        fetch(j0 + 4, bufs[1])
        process(bufs[2])
        return carry

    lax.fori_loop(1, _NTRIPLE + 1, triple, 0)
    process(bufs[0])
    process(bufs[1])
    for buf in bufs:
        pltpu.make_async_copy(buf[4], acc.at[buf[1]], buf[7]).wait()

    plsc.subcore_barrier()
    pltpu.sync_copy(acc.at[pl.ds(sid * _RPT, _RPT)],
                    outp_hbm.at[cid, pl.ds(sid * _RPT, _RPT)])


def _combine_body(p_ref, b_ref, o_ref):
    s = p_ref[0] + p_ref[1]
    num = s[:, :_D]
    den = s[:, _D:_D + 1]
    o_ref[...] = num / (den + 1e-16) + b_ref[...]


def kernel(x, edge_idx, lin_weight, att_dst, att_src, bias):
    n, d = x.shape
    hc = lin_weight.shape[0]
    assert n == _N and d == _D and hc == _D and edge_idx.shape == (2, _E)

    asr = att_src.reshape(1, hc).astype(jnp.float32)
    adt = att_dst.reshape(1, hc).astype(jnp.float32)

    b1 = 1000
    hext, a_src, a_dst = pl.pallas_call(
        _proj_body,
        grid=(n // b1,),
        in_specs=[
            pl.BlockSpec((b1, d), lambda i: (i, 0)),
            pl.BlockSpec((hc, d), lambda i: (0, 0)),
            pl.BlockSpec((1, hc), lambda i: (0, 0)),
            pl.BlockSpec((1, hc), lambda i: (0, 0)),
        ],
        out_specs=[
            pl.BlockSpec((b1, _CP), lambda i: (i, 0)),
            pl.BlockSpec((b1, 1), lambda i: (i, 0)),
            pl.BlockSpec((b1, 1), lambda i: (i, 0)),
        ],
        out_shape=[
            jax.ShapeDtypeStruct((n, _CP), jnp.float32),
            jax.ShapeDtypeStruct((n, 1), jnp.float32),
            jax.ShapeDtypeStruct((n, 1), jnp.float32),
        ],
    )(x, lin_weight, asr, adt)
    a_src = a_src.reshape(n)
    a_dst = a_dst.reshape(n)

    src = edge_idx[0]
    dst = edge_idx[1]
    zeros = jnp.zeros((_RPT, _CP), jnp.float32)

    mesh = plsc.VectorSubcoreMesh(core_axis_name="c", subcore_axis_name="s")
    edge_kernel = functools.partial(
        pl.kernel,
        out_type=jax.ShapeDtypeStruct((_NC, _NP, _CP), jnp.float32),
        mesh=mesh,
        compiler_params=pltpu.CompilerParams(
            needs_layout_passes=False, use_tc_tiling_on_sc=False),
        scratch_types=[
            pltpu.VMEM((_K,), jnp.int32),
            pltpu.VMEM((_K,), jnp.int32),
            pltpu.VMEM((_K,), jnp.float32),
            pltpu.VMEM((_K,), jnp.float32),
            pltpu.VMEM((_K, _CP), jnp.float32),
            pltpu.VMEM((_K,), jnp.int32),
            pltpu.VMEM((_K,), jnp.int32),
            pltpu.VMEM((_K,), jnp.float32),
            pltpu.VMEM((_K,), jnp.float32),
            pltpu.VMEM((_K, _CP), jnp.float32),
            pltpu.VMEM((_K,), jnp.int32),
            pltpu.VMEM((_K,), jnp.int32),
            pltpu.VMEM((_K,), jnp.float32),
            pltpu.VMEM((_K,), jnp.float32),
            pltpu.VMEM((_K, _CP), jnp.float32),
            pltpu.VMEM((_K,), jnp.float32),
            pltpu.VMEM_SHARED((_N,), jnp.float32),
            pltpu.VMEM_SHARED((_N,), jnp.float32),
            pltpu.VMEM_SHARED((_NP, _CP), jnp.float32),
            pltpu.SemaphoreType.DMA,
            pltpu.SemaphoreType.DMA,
            pltpu.SemaphoreType.DMA,
            pltpu.SemaphoreType.DMA,
            pltpu.SemaphoreType.DMA,
            pltpu.SemaphoreType.DMA,
            pltpu.SemaphoreType.DMA,
            pltpu.SemaphoreType.DMA,
            pltpu.SemaphoreType.DMA,
        ],
    )(_edge_body)
    outp = edge_kernel(hext, a_src, a_dst, src, dst, zeros)

    b2 = 1000
    out = pl.pallas_call(
        _combine_body,
        grid=(n // b2,),
        in_specs=[
            pl.BlockSpec((_NC, b2, _CP), lambda i: (0, i, 0)),
            pl.BlockSpec((1, hc), lambda i: (0, 0)),
        ],
        out_specs=pl.BlockSpec((b2, hc), lambda i: (i, 0)),
        out_shape=jax.ShapeDtypeStruct((n, hc), jnp.float32),
    )(outp, bias.reshape(1, hc))
    return out


# speedup vs baseline: 1.3373x; 1.3373x over previous
import functools

import jax
import jax.numpy as jnp
from jax import lax
from jax.experimental import pallas as pl
from jax.experimental.pallas import tpu as pltpu
from jax.experimental.pallas import tpu_sc as plsc

_N = 10000
_E = 320000
_D = 128
_CP = 144
_NC = 2
_NS = 16
_NW = _NC * _NS
_EPW = _E // _NW
_K = 80
_NCHUNK = _EPW // _K
_KB = 25
_NBLK = _NCHUNK // _KB
_SCALE_UNROLL = 4
_NP = 10240
_RPT = _NP // _NS
_LANES = 16


def _proj_body(x_ref, w_ref, as_ref, ad_ref, hext_ref, av_ref, dv_ref):
    x = x_ref[...]
    h = lax.dot_general(x, w_ref[...], (((1,), (1,)), ((), ())),
                        preferred_element_type=jnp.float32)
    b = h.shape[0]
    tail = (lax.broadcasted_iota(jnp.int32, (b, _CP - _D), 1) == 0)
    hext_ref[...] = jnp.concatenate([h, tail.astype(jnp.float32)], axis=1)
    av_ref[...] = jnp.sum(h * as_ref[...], axis=1, keepdims=True)
    dv_ref[...] = jnp.sum(h * ad_ref[...], axis=1, keepdims=True)


def _edge_body(hext_hbm, asrc_hbm, adst_hbm, src4_hbm, dst4_hbm, zero_hbm,
               outp_hbm, sblk0, dblk0, sblk1, dblk1, av0, bv0, rows0, av1, bv1,
               rows1, wbuf, asrc_s, adst_s, acc, sem_r0, sem_v0, sem_s0, sem_r1,
               sem_v1, sem_s1):
    cid = lax.axis_index("c")
    sid = lax.axis_index("s")
    wid = cid * _NS + sid
    pltpu.sync_copy(zero_hbm, acc.at[pl.ds(sid * _RPT, _RPT)])
    @pl.when(sid == 0)
    def _stage_tables():
        pltpu.sync_copy(asrc_hbm, asrc_s)
        pltpu.sync_copy(adst_hbm, adst_s)
    plsc.subcore_barrier()

    marker = (lax.iota(jnp.int32, 16) == 0).astype(jnp.float32)
    IB = ((sblk0, dblk0), (sblk1, dblk1))
    RB = ((av0, bv0, rows0, sem_r0, sem_v0, sem_s0),
          (av1, bv1, rows1, sem_r1, sem_v1, sem_s1))

    def stage(sblock, ib):
        pltpu.sync_copy(src4_hbm.at[wid, sblock], ib[0])
        pltpu.sync_copy(dst4_hbm.at[wid, sblock], ib[1])

    def fetch(c, ib, rb, first=False):
        sb_s, sb_d = ib
        av_c, bv_c, rows, sem_r, sem_v, sem_s = rb
        if not first:
            pltpu.make_async_copy(rows, acc.at[sb_d.at[c]], sem_s).wait()
        pltpu.async_copy(hext_hbm.at[sb_s.at[c]], rows, sem_r)
        pltpu.async_copy(asrc_s.at[sb_s.at[c]], av_c, sem_v)
        pltpu.async_copy(adst_s.at[sb_d.at[c]], bv_c, sem_v)

    def process(c, ib, rb):
        sb_s, sb_d = ib
        av_c, bv_c, rows, sem_r, sem_v, sem_s = rb
        pltpu.make_async_copy(asrc_s.at[sb_s.at[c]], av_c, sem_v).wait()
        pltpu.make_async_copy(adst_s.at[sb_d.at[c]], bv_c, sem_v).wait()
        for g in range(_K // _LANES):
            sl = pl.ds(g * _LANES, _LANES)
            a = av_c[sl] + bv_c[sl]
            a = jnp.where(a >= 0, a, 0.2 * a)
            wbuf[sl] = jnp.exp(a)
        pltpu.make_async_copy(hext_hbm.at[sb_s.at[c]], rows, sem_r).wait()

        def scale(i, c2):
            for u in range(_SCALE_UNROLL):
                e = i * _SCALE_UNROLL + u
                w16 = plsc.load_gather(
                    wbuf, [jnp.full((_LANES,), e, jnp.int32)])
                for v in range(_D // _LANES):
                    sl = pl.ds(v * _LANES, _LANES)
                    rows[e, sl] = rows[e, sl] * w16
                rows[e, pl.ds(_D, _LANES)] = w16 * marker
            return c2

        lax.fori_loop(0, _K // _SCALE_UNROLL, scale, 0)
        pltpu.async_copy(rows, acc.at[sb_d.at[c]], sem_s, add=True)

    stage(0, IB[0])
    fetch(0, IB[0], RB[0], first=True)
    fetch(1, IB[0], RB[1], first=True)
    process(0, IB[0], RB[0])
    fetch(2, IB[0], RB[0])
    process(1, IB[0], RB[1])
    for s in range(_NBLK):
        ib = IB[s % 2]
        pa = s % 2

        def inner(t, carry, ib=ib, pa=pa):
            fetch(2 * t + 1, ib, RB[1 - pa])
            process(2 * t, ib, RB[pa])
            fetch(2 * t + 2, ib, RB[pa])
            process(2 * t + 1, ib, RB[1 - pa])
            return carry

        lax.fori_loop(1 if s == 0 else 0, _KB // 2, inner, 0)
        if s + 1 < _NBLK:
            stage(s + 1, IB[(s + 1) % 2])
            fetch(0, IB[(s + 1) % 2], RB[1 - pa])
        process(_KB - 1, ib, RB[pa])

    for rb in RB:
        pltpu.make_async_copy(rb[2], acc.at[IB[0][1].at[0]], rb[5]).wait()

    plsc.subcore_barrier()
    pltpu.sync_copy(acc.at[pl.ds(sid * _RPT, _RPT)],
                    outp_hbm.at[cid, pl.ds(sid * _RPT, _RPT)])


def _combine_body(p_ref, b_ref, o_ref):
    s = p_ref[0] + p_ref[1]
    num = s[:, :_D]
    den = s[:, _D:_D + 1]
    o_ref[...] = num / (den + 1e-16) + b_ref[...]


def kernel(x, edge_idx, lin_weight, att_dst, att_src, bias):
    n, d = x.shape
    hc = lin_weight.shape[0]
    assert n == _N and d == _D and hc == _D and edge_idx.shape == (2, _E)

    asr = att_src.reshape(1, hc).astype(jnp.float32)
    adt = att_dst.reshape(1, hc).astype(jnp.float32)

    b1 = 1000
    hext, a_src, a_dst = pl.pallas_call(
        _proj_body,
        grid=(n // b1,),
        in_specs=[
            pl.BlockSpec((b1, d), lambda i: (i, 0)),
            pl.BlockSpec((hc, d), lambda i: (0, 0)),
            pl.BlockSpec((1, hc), lambda i: (0, 0)),
            pl.BlockSpec((1, hc), lambda i: (0, 0)),
        ],
        out_specs=[
            pl.BlockSpec((b1, _CP), lambda i: (i, 0)),
            pl.BlockSpec((b1, 1), lambda i: (i, 0)),
            pl.BlockSpec((b1, 1), lambda i: (i, 0)),
        ],
        out_shape=[
            jax.ShapeDtypeStruct((n, _CP), jnp.float32),
            jax.ShapeDtypeStruct((n, 1), jnp.float32),
            jax.ShapeDtypeStruct((n, 1), jnp.float32),
        ],
    )(x, lin_weight, asr, adt)
    a_src = a_src.reshape(n)
    a_dst = a_dst.reshape(n)

    src = edge_idx[0].reshape(_NW, _NBLK, _KB, _K)
    dst = edge_idx[1].reshape(_NW, _NBLK, _KB, _K)
    zeros = jnp.zeros((_RPT, _CP), jnp.float32)

    mesh = plsc.VectorSubcoreMesh(core_axis_name="c", subcore_axis_name="s")
    edge_kernel = functools.partial(
        pl.kernel,
        out_type=jax.ShapeDtypeStruct((_NC, _NP, _CP), jnp.float32),
        mesh=mesh,
        compiler_params=pltpu.CompilerParams(
            needs_layout_passes=False, use_tc_tiling_on_sc=False),
        scratch_types=[
            pltpu.VMEM((_KB, _K), jnp.int32),
            pltpu.VMEM((_KB, _K), jnp.int32),
            pltpu.VMEM((_KB, _K), jnp.int32),
            pltpu.VMEM((_KB, _K), jnp.int32),
            pltpu.VMEM((_K,), jnp.float32),
            pltpu.VMEM((_K,), jnp.float32),
            pltpu.VMEM((_K, _CP), jnp.float32),
            pltpu.VMEM((_K,), jnp.float32),
            pltpu.VMEM((_K,), jnp.float32),
            pltpu.VMEM((_K, _CP), jnp.float32),
            pltpu.VMEM((_K,), jnp.float32),
            pltpu.VMEM_SHARED((_N,), jnp.float32),
            pltpu.VMEM_SHARED((_N,), jnp.float32),
            pltpu.VMEM_SHARED((_NP, _CP), jnp.float32),
            pltpu.SemaphoreType.DMA,
            pltpu.SemaphoreType.DMA,
            pltpu.SemaphoreType.DMA,
            pltpu.SemaphoreType.DMA,
            pltpu.SemaphoreType.DMA,
            pltpu.SemaphoreType.DMA,
        ],
    )(_edge_body)
    outp = edge_kernel(hext, a_src, a_dst, src, dst, zeros)

    b2 = 1000
    out = pl.pallas_call(
        _combine_body,
        grid=(n // b2,),
        in_specs=[
            pl.BlockSpec((_NC, b2, _CP), lambda i: (0, i, 0)),
            pl.BlockSpec((1, hc), lambda i: (0, 0)),
        ],
        out_specs=pl.BlockSpec((b2, hc), lambda i: (i, 0)),
        out_shape=jax.ShapeDtypeStruct((n, hc), jnp.float32),
    )(outp, bias.reshape(1, hc))
    return out


# speedup vs baseline: 1.3420x; 1.0035x over previous
import functools

import jax
import jax.numpy as jnp
from jax import lax
from jax.experimental import pallas as pl
from jax.experimental.pallas import tpu as pltpu
from jax.experimental.pallas import tpu_sc as plsc

_N = 10000
_E = 320000
_D = 128
_CP = 144
_NC = 2
_NS = 16
_NW = _NC * _NS
_EPW = _E // _NW
_K = 80
_NCHUNK = _EPW // _K
_KB = 25
_NBLK = _NCHUNK // _KB
_SCALE_UNROLL = 4
_NP = 10240
_RPT = _NP // _NS
_LANES = 16


def _proj_body(x_ref, w_ref, as_ref, ad_ref, hext_ref, dv_ref):
    x = x_ref[...]
    h = lax.dot_general(x, w_ref[...], (((1,), (1,)), ((), ())),
                        preferred_element_type=jnp.float32)
    b = h.shape[0]
    av = jnp.sum(h * as_ref[...], axis=1, keepdims=True)
    ones = jnp.ones((b, 1), jnp.float32)
    pad = jnp.zeros((b, _CP - _D - 2), jnp.float32)
    hext_ref[...] = jnp.concatenate([h, ones, av, pad], axis=1)
    dv_ref[...] = jnp.sum(h * ad_ref[...], axis=1, keepdims=True)


def _edge_body(hext_hbm, adst_hbm, src4_hbm, dst4_hbm, zero_hbm,
               outp_hbm, sblk0, dblk0, sblk1, dblk1, bv0, rows0, bv1,
               rows1, wbuf, adst_s, acc, sem_r0, sem_v0, sem_s0, sem_r1,
               sem_v1, sem_s1):
    cid = lax.axis_index("c")
    sid = lax.axis_index("s")
    wid = cid * _NS + sid
    pltpu.sync_copy(zero_hbm, acc.at[pl.ds(sid * _RPT, _RPT)])
    @pl.when(sid == 0)
    def _stage_tables():
        pltpu.sync_copy(adst_hbm, adst_s)
    plsc.subcore_barrier()

    marker = (lax.iota(jnp.int32, 16) == 0).astype(jnp.float32)
    IB = ((sblk0, dblk0), (sblk1, dblk1))
    RB = ((bv0, rows0, sem_r0, sem_v0, sem_s0),
          (bv1, rows1, sem_r1, sem_v1, sem_s1))

    def stage(sblock, ib):
        pltpu.sync_copy(src4_hbm.at[wid, sblock], ib[0])
        pltpu.sync_copy(dst4_hbm.at[wid, sblock], ib[1])

    def fetch(c, ib, rb, first=False):
        sb_s, sb_d = ib
        bv_c, rows, sem_r, sem_v, sem_s = rb
        if not first:
            pltpu.make_async_copy(rows, acc.at[sb_d.at[c]], sem_s).wait()
        pltpu.async_copy(hext_hbm.at[sb_s.at[c]], rows, sem_r)
        pltpu.async_copy(adst_s.at[sb_d.at[c]], bv_c, sem_v)

    def process(c, ib, rb):
        sb_s, sb_d = ib
        bv_c, rows, sem_r, sem_v, sem_s = rb
        pltpu.make_async_copy(adst_s.at[sb_d.at[c]], bv_c, sem_v).wait()
        pltpu.make_async_copy(hext_hbm.at[sb_s.at[c]], rows, sem_r).wait()
        for g in range(_K // _LANES):
            sl = pl.ds(g * _LANES, _LANES)
            e16 = lax.iota(jnp.int32, _LANES) + g * _LANES
            asrc16 = plsc.load_gather(
                rows, [e16, jnp.full((_LANES,), _D + 1, jnp.int32)])
            a = asrc16 + bv_c[sl]
            a = jnp.where(a >= 0, a, 0.2 * a)
            wbuf[sl] = jnp.exp(a)

        def scale(i, c2):
            for u in range(_SCALE_UNROLL):
                e = i * _SCALE_UNROLL + u
                w16 = plsc.load_gather(
                    wbuf, [jnp.full((_LANES,), e, jnp.int32)])
                for v in range(_D // _LANES):
                    sl = pl.ds(v * _LANES, _LANES)
                    rows[e, sl] = rows[e, sl] * w16
                rows[e, pl.ds(_D, _LANES)] = w16 * marker
            return c2

        lax.fori_loop(0, _K // _SCALE_UNROLL, scale, 0)
        pltpu.async_copy(rows, acc.at[sb_d.at[c]], sem_s, add=True)

    stage(0, IB[0])
    fetch(0, IB[0], RB[0], first=True)
    fetch(1, IB[0], RB[1], first=True)
    process(0, IB[0], RB[0])
    fetch(2, IB[0], RB[0])
    process(1, IB[0], RB[1])
    for s in range(_NBLK):
        ib = IB[s % 2]
        pa = s % 2

        def inner(t, carry, ib=ib, pa=pa):
            fetch(2 * t + 1, ib, RB[1 - pa])
            process(2 * t, ib, RB[pa])
            fetch(2 * t + 2, ib, RB[pa])
            process(2 * t + 1, ib, RB[1 - pa])
            return carry

        lax.fori_loop(1 if s == 0 else 0, _KB // 2, inner, 0)
        if s + 1 < _NBLK:
            stage(s + 1, IB[(s + 1) % 2])
            fetch(0, IB[(s + 1) % 2], RB[1 - pa])
        process(_KB - 1, ib, RB[pa])

    for rb in RB:
        pltpu.make_async_copy(rb[1], acc.at[IB[0][1].at[0]], rb[4]).wait()

    plsc.subcore_barrier()
    pltpu.sync_copy(acc.at[pl.ds(sid * _RPT, _RPT)],
                    outp_hbm.at[cid, pl.ds(sid * _RPT, _RPT)])


def _combine_body(p_ref, b_ref, o_ref):
    s = p_ref[0] + p_ref[1]
    num = s[:, :_D]
    den = s[:, _D:_D + 1]
    o_ref[...] = num / (den + 1e-16) + b_ref[...]


def kernel(x, edge_idx, lin_weight, att_dst, att_src, bias):
    n, d = x.shape
    hc = lin_weight.shape[0]
    assert n == _N and d == _D and hc == _D and edge_idx.shape == (2, _E)

    asr = att_src.reshape(1, hc).astype(jnp.float32)
    adt = att_dst.reshape(1, hc).astype(jnp.float32)

    b1 = 1000
    hext, a_dst = pl.pallas_call(
        _proj_body,
        grid=(n // b1,),
        in_specs=[
            pl.BlockSpec((b1, d), lambda i: (i, 0)),
            pl.BlockSpec((hc, d), lambda i: (0, 0)),
            pl.BlockSpec((1, hc), lambda i: (0, 0)),
            pl.BlockSpec((1, hc), lambda i: (0, 0)),
        ],
        out_specs=[
            pl.BlockSpec((b1, _CP), lambda i: (i, 0)),
            pl.BlockSpec((b1, 1), lambda i: (i, 0)),
        ],
        out_shape=[
            jax.ShapeDtypeStruct((n, _CP), jnp.float32),
            jax.ShapeDtypeStruct((n, 1), jnp.float32),
        ],
    )(x, lin_weight, asr, adt)
    a_dst = a_dst.reshape(n)

    src = edge_idx[0].reshape(_NW, _NBLK, _KB, _K)
    dst = edge_idx[1].reshape(_NW, _NBLK, _KB, _K)
    zeros = jnp.zeros((_RPT, _CP), jnp.float32)

    mesh = plsc.VectorSubcoreMesh(core_axis_name="c", subcore_axis_name="s")
    edge_kernel = functools.partial(
        pl.kernel,
        out_type=jax.ShapeDtypeStruct((_NC, _NP, _CP), jnp.float32),
        mesh=mesh,
        compiler_params=pltpu.CompilerParams(
            needs_layout_passes=False, use_tc_tiling_on_sc=False),
        scratch_types=[
            pltpu.VMEM((_KB, _K), jnp.int32),
            pltpu.VMEM((_KB, _K), jnp.int32),
            pltpu.VMEM((_KB, _K), jnp.int32),
            pltpu.VMEM((_KB, _K), jnp.int32),
            pltpu.VMEM((_K,), jnp.float32),
            pltpu.VMEM((_K, _CP), jnp.float32),
            pltpu.VMEM((_K,), jnp.float32),
            pltpu.VMEM((_K, _CP), jnp.float32),
            pltpu.VMEM((_K,), jnp.float32),
            pltpu.VMEM_SHARED((_N,), jnp.float32),
            pltpu.VMEM_SHARED((_NP, _CP), jnp.float32),
            pltpu.SemaphoreType.DMA,
            pltpu.SemaphoreType.DMA,
            pltpu.SemaphoreType.DMA,
            pltpu.SemaphoreType.DMA,
            pltpu.SemaphoreType.DMA,
            pltpu.SemaphoreType.DMA,
        ],
    )(_edge_body)
    outp = edge_kernel(hext, a_dst, src, dst, zeros)

    b2 = 1000
    out = pl.pallas_call(
        _combine_body,
        grid=(n // b2,),
        in_specs=[
            pl.BlockSpec((_NC, b2, _CP), lambda i: (0, i, 0)),
            pl.BlockSpec((1, hc), lambda i: (0, 0)),
        ],
        out_specs=pl.BlockSpec((b2, hc), lambda i: (i, 0)),
        out_shape=jax.ShapeDtypeStruct((n, hc), jnp.float32),
    )(outp, bias.reshape(1, hc))
    return out
